# baseline (speedup 1.0000x reference)
import functools

import jax
import jax.numpy as jnp
import numpy as np
from jax import lax
from jax.experimental import pallas as pl
from jax.experimental.pallas import tpu as pltpu
from jax.experimental.pallas import tpu_sc as plsc

STRIDE = 4
NUM_CLASSES = 80
OUT_H = 128
OUT_W = 128
B = 8
N_BOXES = 32
R_MAX = 15

_DENOMS = np.asarray(
    [np.float32(2.0 * (r / 3 + 1 / 6) ** 2) for r in range(R_MAX + 1)], np.float32
)
_EPS = np.float32(np.finfo(np.float32).eps)
_NUMEL = float(B * NUM_CLASSES * OUT_H * OUT_W)

N_WORKERS = 32
BOX_PER_W = (B * N_BOXES) // N_WORKERS

_I_SLOT, _I_RS, _I_CX, _I_CY, _I_RX, _I_RY = range(6)
_F_DENX, _F_DENY = range(2)


def _tc_body(ints_ref, flts_ref, hm_ref, out_ref, t_ref):
    b = pl.program_id(0)

    def _ssq_step(c, acc):
        x = hm_ref[0, pl.ds(c * 8, 8)]
        return acc + jnp.sum(x * x, axis=0)

    ssq_vec = lax.fori_loop(
        0, NUM_CLASSES // 8, _ssq_step, jnp.zeros((OUT_H, OUT_W), jnp.float32)
    )
    sumsq = jnp.sum(ssq_vec)

    def _zero_step(j, _):
        t_ref[j] = jnp.zeros((OUT_H, OUT_W), jnp.float32)
        return 0

    lax.fori_loop(0, N_BOXES, _zero_step, 0)

    row_iota = lax.broadcasted_iota(jnp.int32, (40, OUT_W), 0)
    col_iota = lax.broadcasted_iota(jnp.int32, (40, OUT_W), 1)
    for i in range(N_BOXES):
        slot = ints_ref[b, i, _I_SLOT]
        rs = ints_ref[b, i, _I_RS]
        cx = ints_ref[b, i, _I_CX]
        cy = ints_ref[b, i, _I_CY]
        rx = ints_ref[b, i, _I_RX]
        ry = ints_ref[b, i, _I_RY]
        denx = flts_ref[b, i, _F_DENX]
        deny = flts_ref[b, i, _F_DENY]
        dy = (rs + row_iota) - cy
        dx = col_iota - cx
        e = dx.astype(jnp.float32) ** 2 / denx + dy.astype(jnp.float32) ** 2 / deny
        g = jnp.exp(-e)
        g = jnp.where(g < _EPS, jnp.float32(0.0), g)
        mask = (jnp.abs(dx) <= rx) & (jnp.abs(dy) <= ry)
        vals = jnp.where(mask, g, jnp.float32(0.0))
        cur = t_ref[slot, pl.ds(rs, 40), :]
        t_ref[slot, pl.ds(rs, 40), :] = jnp.maximum(cur, vals)

    def _corr_step(j, acc):
        lab = ints_ref[b, j, _I_SLOT + 6]
        tj = t_ref[j]
        hj = hm_ref[0, lab]
        return acc + tj * (tj - 2.0 * hj)

    corr_vec = lax.fori_loop(
        0, N_BOXES, _corr_step, jnp.zeros((OUT_H, OUT_W), jnp.float32)
    )
    corr = jnp.sum(corr_vec)

    lane = lax.broadcasted_iota(jnp.int32, (1, 128), 1)
    row = jnp.where(lane == 0, sumsq, jnp.float32(0.0)) + jnp.where(
        lane == 1, corr, jnp.float32(0.0)
    )
    out_ref[0, 0] = row[0]


def _sc_body(
    b2flat, idxs, mavs, out_hbm,
    idx_v, gath_v, ma_v, out_v, sem,
):
    w = lax.axis_index("s") * 2 + lax.axis_index("c")
    pltpu.sync_copy(idxs.at[w], idx_v)
    cps = [
        pltpu.async_copy(b2flat.at[idx_v.at[r]], gath_v.at[r], sem)
        for r in range(4)
    ]
    pltpu.sync_copy(mavs.at[w], ma_v)
    for cp in cps:
        cp.wait()
    four = jnp.full((16,), 4.0, jnp.float32)
    acc = jnp.zeros((16,), jnp.float32)
    cnt = jnp.zeros((16,), jnp.float32)
    for k in range(BOX_PER_W):
        mf = ma_v[k, 4]
        d = jnp.zeros((16,), jnp.float32)
        for c in range(4):
            off = gath_v[k // 2, pl.ds((k % 2) * 64 + c * 16, 16)]
            if c < 2:
                d = d + jnp.abs(ma_v[k, c] - four * off)
            else:
                d = d + jnp.abs(ma_v[k, c] + four * off)
        acc = acc + d * mf
        cnt = cnt + mf
    out_v[0] = acc
    out_v[1] = cnt
    pltpu.sync_copy(out_v, out_hbm.at[w])


def kernel(heatmap, box_2d, boxes, labels):
    x = boxes[..., 0]
    y = boxes[..., 1]
    w = boxes[..., 2]
    h = boxes[..., 3]
    xs, ys, ws, hs = x / STRIDE, y / STRIDE, w / STRIDE, h / STRIDE
    cx = jnp.round(xs + ws / 2).astype(jnp.int32)
    cy = jnp.round(ys + hs / 2).astype(jnp.int32)
    rx = jnp.minimum(jnp.maximum(0, jnp.round(ws / 2 * 0.5).astype(jnp.int32)), R_MAX)
    ry = jnp.minimum(jnp.maximum(0, jnp.round(hs / 2 * 0.5).astype(jnp.int32)), R_MAX)
    table = jnp.asarray(_DENOMS)
    denx = table[rx]
    deny = table[ry]
    eq = labels[:, :, None] == labels[:, None, :]
    slot = jnp.argmax(eq, axis=-1).astype(jnp.int32)
    rs = jnp.clip(8 * ((cy - R_MAX) // 8), 0, OUT_H - 40).astype(jnp.int32)

    ints = jnp.stack([slot, rs, cx, cy, rx, ry, labels], axis=-1).astype(jnp.int32)
    flts = jnp.stack([denx, deny], axis=-1).astype(jnp.float32)

    j16 = np.arange(16, dtype=np.int32)
    dxj = jnp.asarray(np.minimum(j16 // 3, 4) - 1, jnp.int32)
    dyj = jnp.asarray(j16 % 3 - 1, jnp.int32)
    ncx = cx[..., None] + dxj
    ncy = cy[..., None] + dyj
    inb = (
        (ncx >= 0) & (ncx < OUT_W) & (ncy >= 0) & (ncy < OUT_H)
        & (jnp.asarray(j16 < 9)[None, None, :])
    )
    xyxy = jnp.stack([x, y, x + w, y + h], axis=-1)
    ncxf = ncx.astype(jnp.float32) * STRIDE
    ncyf = ncy.astype(jnp.float32) * STRIDE
    mavs = jnp.stack(
        [
            ncxf - xyxy[..., 0:1],
            ncyf - xyxy[..., 1:2],
            ncxf - xyxy[..., 2:3],
            ncyf - xyxy[..., 3:4],
            inb.astype(jnp.float32),
        ],
        axis=-2,
    ).reshape(N_WORKERS, BOX_PER_W, 5, 16)
    rowyc = jnp.clip(ncy, 0, OUT_H - 1)
    colxc = jnp.clip(ncx, 0, OUT_W - 1)
    bb = jnp.arange(B, dtype=jnp.int32)[:, None, None, None]
    cc = jnp.arange(4, dtype=jnp.int32)[None, None, :, None]
    idxs = ((bb * 4 + cc) * OUT_H + rowyc[:, :, None, :]) * OUT_W + colxc[
        :, :, None, :
    ]
    idxs = idxs.astype(jnp.int32).reshape(N_WORKERS, 4, 128)

    b2flat = box_2d.reshape(B * 4 * OUT_H * OUT_W)

    sc_out = pl.kernel(
        _sc_body,
        out_type=jax.ShapeDtypeStruct((N_WORKERS, 2, 16), jnp.float32),
        mesh=plsc.VectorSubcoreMesh(core_axis_name="c", subcore_axis_name="s"),
        scratch_types=[
            pltpu.VMEM((4, 128), jnp.int32),
            pltpu.VMEM((4, 128), jnp.float32),
            pltpu.VMEM((BOX_PER_W, 5, 16), jnp.float32),
            pltpu.VMEM((2, 16), jnp.float32),
            pltpu.SemaphoreType.DMA,
        ],
    )(b2flat, idxs, mavs)

    tc_out = pl.pallas_call(
        _tc_body,
        grid=(B,),
        in_specs=[
            pl.BlockSpec(memory_space=pltpu.SMEM),
            pl.BlockSpec(memory_space=pltpu.SMEM),
            pl.BlockSpec((1, NUM_CLASSES, OUT_H, OUT_W), lambda b: (b, 0, 0, 0)),
        ],
        out_specs=pl.BlockSpec((1, 1, 128), lambda b: (b, 0, 0)),
        out_shape=jax.ShapeDtypeStruct((B, 1, 128), jnp.float32),
        scratch_shapes=[pltpu.VMEM((N_BOXES, OUT_H, OUT_W), jnp.float32)],
        compiler_params=pltpu.CompilerParams(
            dimension_semantics=("parallel",),
        ),
    )(ints, flts, heatmap)

    hm_loss = (jnp.sum(tc_out[:, 0, 0]) + jnp.sum(tc_out[:, 0, 1])) / jnp.float32(
        _NUMEL
    )
    diff_b = jnp.sum(sc_out[:, 0].reshape(B, 4 * 16), axis=-1)
    cnt_b = jnp.sum(sc_out[:, 1].reshape(B, 4 * 16), axis=-1)
    box_loss = jnp.mean(diff_b / (cnt_b * jnp.float32(4.0)))
    return jnp.stack([hm_loss, box_loss])


# speedup vs baseline: 1.2522x; 1.2522x over previous
import jax
import jax.numpy as jnp
import numpy as np
from jax import lax
from jax.experimental import pallas as pl
from jax.experimental.pallas import tpu as pltpu
from jax.experimental.pallas import tpu_sc as plsc

STRIDE = 4
NUM_CLASSES = 80
OUT_H = 128
OUT_W = 128
B = 8
N_BOXES = 32
R_MAX = 15

_DENOMS = [np.float32(2.0 * (r / 3 + 1 / 6) ** 2) for r in range(R_MAX + 1)]
_EPS = np.float32(np.finfo(np.float32).eps)
_NUMEL = float(B * NUM_CLASSES * OUT_H * OUT_W)

N_WORKERS = 32
BOX_PER_W = (B * N_BOXES) // N_WORKERS

_I_SLOT, _I_RS, _I_CX, _I_CY, _I_RX, _I_RY, _I_LAB = range(7)
_F_DENX, _F_DENY = range(2)


def _prep_body(boxes_ref, labels_ref, ints_ref, flts_ref, idxs_ref, mavs_ref):
    x = boxes_ref[0]
    y = boxes_ref[1]
    w = boxes_ref[2]
    h = boxes_ref[3]
    xs, ys, ws, hs = x / STRIDE, y / STRIDE, w / STRIDE, h / STRIDE
    cxf = jnp.round(xs + ws / 2)
    cyf = jnp.round(ys + hs / 2)
    cx = cxf.astype(jnp.int32)
    cy = cyf.astype(jnp.int32)
    rx = jnp.minimum(jnp.maximum(0, jnp.round(ws / 4).astype(jnp.int32)), R_MAX)
    ry = jnp.minimum(jnp.maximum(0, jnp.round(hs / 4).astype(jnp.int32)), R_MAX)
    denx = jnp.full(rx.shape, _DENOMS[0], jnp.float32)
    deny = jnp.full(ry.shape, _DENOMS[0], jnp.float32)
    for r in range(1, R_MAX + 1):
        denx = jnp.where(rx == r, _DENOMS[r], denx)
        deny = jnp.where(ry == r, _DENOMS[r], deny)
    lab = labels_ref[...]
    eq = lab[:, :, None] == lab[:, None, :]
    jidx = lax.broadcasted_iota(jnp.int32, (B, N_BOXES, N_BOXES), 2)
    slot = jnp.min(jnp.where(eq, jidx, N_BOXES), axis=-1)
    rs = jnp.clip(8 * ((cy - R_MAX) // 8), 0, OUT_H - 40)

    ints_ref[_I_SLOT] = slot
    ints_ref[_I_RS] = rs
    ints_ref[_I_CX] = cx
    ints_ref[_I_CY] = cy
    ints_ref[_I_RX] = rx
    ints_ref[_I_RY] = ry
    ints_ref[_I_LAB] = lab
    flts_ref[_F_DENX] = denx
    flts_ref[_F_DENY] = deny

    j16 = lax.broadcasted_iota(jnp.int32, (16,), 0)
    dxj = jnp.minimum(j16 // 3, 4) - 1
    dyj = j16 % 3 - 1
    ncx = cx[:, :, None] + dxj
    ncy = cy[:, :, None] + dyj
    inb = (
        (ncx >= 0) & (ncx < OUT_W) & (ncy >= 0) & (ncy < OUT_H) & (j16 < 9)
    )
    rowyc = jnp.clip(ncy, 0, OUT_H - 1)
    colxc = jnp.clip(ncx, 0, OUT_W - 1)
    bcst = lax.broadcasted_iota(jnp.int32, (B, N_BOXES, 4, 16), 0)
    ccst = lax.broadcasted_iota(jnp.int32, (B, N_BOXES, 4, 16), 2)
    idxs_ref[...] = ((bcst * 4 + ccst) * OUT_H + rowyc[:, :, None, :]) * OUT_W + (
        colxc[:, :, None, :]
    )
    ncxf = ncx.astype(jnp.float32) * STRIDE
    ncyf = ncy.astype(jnp.float32) * STRIDE
    mavs_ref[...] = jnp.stack(
        [
            ncxf - x[:, :, None],
            ncyf - y[:, :, None],
            ncxf - (x + w)[:, :, None],
            ncyf - (y + h)[:, :, None],
            inb.astype(jnp.float32),
        ],
        axis=-2,
    )


def _sc_body(b2flat, idxs, mavs, out_hbm, idx_v, gath_v, ma_v, out_v, sem):
    w = lax.axis_index("s") * 2 + lax.axis_index("c")
    pltpu.sync_copy(idxs.at[w], idx_v)
    cps = [
        pltpu.async_copy(b2flat.at[idx_v.at[r]], gath_v.at[r], sem)
        for r in range(4)
    ]
    pltpu.sync_copy(mavs.at[w], ma_v)
    for cp in cps:
        cp.wait()
    four = jnp.full((16,), 4.0, jnp.float32)
    acc = jnp.zeros((16,), jnp.float32)
    cnt = jnp.zeros((16,), jnp.float32)
    for k in range(BOX_PER_W):
        mf = ma_v[k, 4]
        d = jnp.zeros((16,), jnp.float32)
        for c in range(4):
            off = gath_v[k // 2, pl.ds((k % 2) * 64 + c * 16, 16)]
            if c < 2:
                d = d + jnp.abs(ma_v[k, c] - four * off)
            else:
                d = d + jnp.abs(ma_v[k, c] + four * off)
        acc = acc + d * mf
        cnt = cnt + mf
    out_v[0] = acc
    out_v[1] = cnt
    pltpu.sync_copy(out_v, out_hbm.at[w])


def _tc_body(ints_ref, flts_ref, hm_ref, scv_ref, out_ref, t_ref):
    b = pl.program_id(0)

    def _ssq_step(c, acc):
        xc = hm_ref[0, pl.ds(c * 8, 8)]
        return acc + jnp.sum(xc * xc, axis=0)

    ssq_vec = lax.fori_loop(
        0, NUM_CLASSES // 8, _ssq_step, jnp.zeros((OUT_H, OUT_W), jnp.float32)
    )
    sumsq = jnp.sum(ssq_vec)

    def _zero_step(j, _):
        t_ref[j] = jnp.zeros((OUT_H, OUT_W), jnp.float32)
        return 0

    lax.fori_loop(0, N_BOXES, _zero_step, 0)

    row_iota = lax.broadcasted_iota(jnp.int32, (40, OUT_W), 0)
    col_iota = lax.broadcasted_iota(jnp.int32, (40, OUT_W), 1)
    for i in range(N_BOXES):
        slot = ints_ref[_I_SLOT, b, i]
        rs = ints_ref[_I_RS, b, i]
        cx = ints_ref[_I_CX, b, i]
        cy = ints_ref[_I_CY, b, i]
        rx = ints_ref[_I_RX, b, i]
        ry = ints_ref[_I_RY, b, i]
        denx = flts_ref[_F_DENX, b, i]
        deny = flts_ref[_F_DENY, b, i]
        dy = (rs + row_iota) - cy
        dx = col_iota - cx
        e = dx.astype(jnp.float32) ** 2 / denx + dy.astype(jnp.float32) ** 2 / deny
        g = jnp.exp(-e)
        g = jnp.where(g < _EPS, jnp.float32(0.0), g)
        mask = (jnp.abs(dx) <= rx) & (jnp.abs(dy) <= ry)
        vals = jnp.where(mask, g, jnp.float32(0.0))
        cur = t_ref[slot, pl.ds(rs, 40), :]
        t_ref[slot, pl.ds(rs, 40), :] = jnp.maximum(cur, vals)

    def _corr_step(j, acc):
        lab = ints_ref[_I_LAB, b, j]
        tj = t_ref[j]
        hj = hm_ref[0, lab]
        return acc + tj * (tj - 2.0 * hj)

    corr_vec = lax.fori_loop(
        0, N_BOXES, _corr_step, jnp.zeros((OUT_H, OUT_W), jnp.float32)
    )
    corr = jnp.sum(corr_vec)

    lane = lax.broadcasted_iota(jnp.int32, (1, 128), 1)
    row = jnp.where(lane == 0, sumsq + corr, jnp.float32(0.0))

    @pl.when(b == 0)
    def _():
        out_ref[0, 0] = row[0]

    @pl.when(b > 0)
    def _():
        out_ref[0, 0] = out_ref[0, 0] + row[0]

    @pl.when(b == B - 1)
    def _():
        tot = out_ref[0, 0]
        hm_loss = jnp.sum(jnp.where(lane[0] == 0, tot, 0.0)) / jnp.float32(_NUMEL)
        bl = jnp.float32(0.0)
        for b8 in range(B):
            dsum = jnp.sum(scv_ref[pl.ds(4 * b8, 4), 0])
            csum = jnp.sum(scv_ref[pl.ds(4 * b8, 4), 1])
            bl = bl + dsum / (csum * jnp.float32(4.0))
        box_loss = bl / jnp.float32(B)
        out_ref[0, 0] = jnp.where(lane[0] == 0, hm_loss, 0.0) + jnp.where(
            lane[0] == 1, box_loss, 0.0
        )


def kernel(heatmap, box_2d, boxes, labels):
    boxes_t = jnp.transpose(boxes, (2, 0, 1))
    ints, flts, idxs4, mavs4 = pl.pallas_call(
        _prep_body,
        out_shape=[
            jax.ShapeDtypeStruct((7, B, N_BOXES), jnp.int32),
            jax.ShapeDtypeStruct((2, B, N_BOXES), jnp.float32),
            jax.ShapeDtypeStruct((B, N_BOXES, 4, 16), jnp.int32),
            jax.ShapeDtypeStruct((B, N_BOXES, 5, 16), jnp.float32),
        ],
    )(boxes_t, labels)

    idxs = idxs4.reshape(N_WORKERS, 4, 128)
    mavs = mavs4.reshape(N_WORKERS, BOX_PER_W, 5, 16)
    b2flat = box_2d.reshape(B * 4 * OUT_H * OUT_W)

    sc_out = pl.kernel(
        _sc_body,
        out_type=jax.ShapeDtypeStruct((N_WORKERS, 2, 16), jnp.float32),
        mesh=plsc.VectorSubcoreMesh(core_axis_name="c", subcore_axis_name="s"),
        scratch_types=[
            pltpu.VMEM((4, 128), jnp.int32),
            pltpu.VMEM((4, 128), jnp.float32),
            pltpu.VMEM((BOX_PER_W, 5, 16), jnp.float32),
            pltpu.VMEM((2, 16), jnp.float32),
            pltpu.SemaphoreType.DMA,
        ],
    )(b2flat, idxs, mavs)

    tc_out = pl.pallas_call(
        _tc_body,
        grid=(B,),
        in_specs=[
            pl.BlockSpec(memory_space=pltpu.SMEM),
            pl.BlockSpec(memory_space=pltpu.SMEM),
            pl.BlockSpec((1, NUM_CLASSES, OUT_H, OUT_W), lambda b: (b, 0, 0, 0)),
            pl.BlockSpec((N_WORKERS, 2, 16), lambda b: (0, 0, 0)),
        ],
        out_specs=pl.BlockSpec((1, 1, 128), lambda b: (0, 0, 0)),
        out_shape=jax.ShapeDtypeStruct((1, 1, 128), jnp.float32),
        scratch_shapes=[pltpu.VMEM((N_BOXES, OUT_H, OUT_W), jnp.float32)],
        compiler_params=pltpu.CompilerParams(
            dimension_semantics=("arbitrary",),
        ),
    )(ints, flts, heatmap, sc_out)

    return jnp.stack([tc_out[0, 0, 0], tc_out[0, 0, 1]])


# speedup vs baseline: 1.3211x; 1.0550x over previous
import jax
import jax.numpy as jnp
import numpy as np
from jax import lax
from jax.experimental import pallas as pl
from jax.experimental.pallas import tpu as pltpu
from jax.experimental.pallas import tpu_sc as plsc

STRIDE = 4
NUM_CLASSES = 80
OUT_H = 128
OUT_W = 128
B = 8
N_BOXES = 32
R_MAX = 15

_DENOMS = [np.float32(2.0 * (r / 3 + 1 / 6) ** 2) for r in range(R_MAX + 1)]
_EPS = np.float32(np.finfo(np.float32).eps)
_NUMEL = float(B * NUM_CLASSES * OUT_H * OUT_W)

N_WORKERS = 32
BOX_PER_W = (B * N_BOXES) // N_WORKERS

_I_SLOT, _I_RS, _I_CX, _I_CY, _I_RX, _I_RY, _I_LAB = range(7)
_F_DENX, _F_DENY = range(2)


def _prep_body(boxes_ref, labels_ref, ints_ref, flts_ref, idxs_ref, mavs_ref):
    x = boxes_ref[:, :, 0]
    y = boxes_ref[:, :, 1]
    w = boxes_ref[:, :, 2]
    h = boxes_ref[:, :, 3]
    xs, ys, ws, hs = x / STRIDE, y / STRIDE, w / STRIDE, h / STRIDE
    cxf = jnp.round(xs + ws / 2)
    cyf = jnp.round(ys + hs / 2)
    cx = cxf.astype(jnp.int32)
    cy = cyf.astype(jnp.int32)
    rx = jnp.minimum(jnp.maximum(0, jnp.round(ws / 4).astype(jnp.int32)), R_MAX)
    ry = jnp.minimum(jnp.maximum(0, jnp.round(hs / 4).astype(jnp.int32)), R_MAX)
    denx = jnp.full(rx.shape, _DENOMS[0], jnp.float32)
    deny = jnp.full(ry.shape, _DENOMS[0], jnp.float32)
    for r in range(1, R_MAX + 1):
        denx = jnp.where(rx == r, _DENOMS[r], denx)
        deny = jnp.where(ry == r, _DENOMS[r], deny)
    lab = labels_ref[...]
    eq = lab[:, :, None] == lab[:, None, :]
    jidx = lax.broadcasted_iota(jnp.int32, (B, N_BOXES, N_BOXES), 2)
    slot = jnp.min(jnp.where(eq, jidx, N_BOXES), axis=-1)
    rs = jnp.clip(8 * ((cy - R_MAX) // 8), 0, OUT_H - 40)

    ints_ref[_I_SLOT] = slot
    ints_ref[_I_RS] = rs
    ints_ref[_I_CX] = cx
    ints_ref[_I_CY] = cy
    ints_ref[_I_RX] = rx
    ints_ref[_I_RY] = ry
    ints_ref[_I_LAB] = lab
    flts_ref[_F_DENX] = denx
    flts_ref[_F_DENY] = deny

    j16 = lax.broadcasted_iota(jnp.int32, (16,), 0)
    dxj = jnp.minimum(j16 // 3, 4) - 1
    dyj = j16 % 3 - 1
    ncx = cx[:, :, None] + dxj
    ncy = cy[:, :, None] + dyj
    inb = (
        (ncx >= 0) & (ncx < OUT_W) & (ncy >= 0) & (ncy < OUT_H) & (j16 < 9)
    )
    rowyc = jnp.clip(ncy, 0, OUT_H - 1)
    colxc = jnp.clip(ncx, 0, OUT_W - 1)
    bcst = lax.broadcasted_iota(jnp.int32, (B, N_BOXES, 4, 16), 0)
    ccst = lax.broadcasted_iota(jnp.int32, (B, N_BOXES, 4, 16), 2)
    idxs_ref[...] = ((bcst * 4 + ccst) * OUT_H + rowyc[:, :, None, :]) * OUT_W + (
        colxc[:, :, None, :]
    )
    ncxf = ncx.astype(jnp.float32) * STRIDE
    ncyf = ncy.astype(jnp.float32) * STRIDE
    mavs_ref[...] = jnp.stack(
        [
            ncxf - x[:, :, None],
            ncyf - y[:, :, None],
            ncxf - (x + w)[:, :, None],
            ncyf - (y + h)[:, :, None],
            inb.astype(jnp.float32),
        ],
        axis=-2,
    )


def _sc_body(b2flat, idxs, mavs, out_hbm, idx_v, gath_v, ma_v, out_v, sem):
    w = lax.axis_index("s") * 2 + lax.axis_index("c")
    pltpu.sync_copy(idxs.at[w], idx_v)
    cps = [
        pltpu.async_copy(b2flat.at[idx_v.at[r]], gath_v.at[r], sem)
        for r in range(4)
    ]
    pltpu.sync_copy(mavs.at[w], ma_v)
    for cp in cps:
        cp.wait()
    four = jnp.full((16,), 4.0, jnp.float32)

    def _box_step(k, carry):
        acc, cnt = carry
        mf = ma_v[k, 4]
        d = jnp.zeros((16,), jnp.float32)
        for c in range(4):
            off = gath_v[k // 2, pl.ds((k % 2) * 64 + c * 16, 16)]
            if c < 2:
                d = d + jnp.abs(ma_v[k, c] - four * off)
            else:
                d = d + jnp.abs(ma_v[k, c] + four * off)
        return acc + d * mf, cnt + mf

    acc, cnt = lax.fori_loop(
        0,
        BOX_PER_W,
        _box_step,
        (jnp.zeros((16,), jnp.float32), jnp.zeros((16,), jnp.float32)),
    )
    out_v[0] = acc
    out_v[1] = cnt
    pltpu.sync_copy(out_v, out_hbm.at[w])


def _tc_body(ints_ref, flts_ref, hm_ref, scv_ref, out_ref, t_ref):
    b = pl.program_id(0)

    def _ssq_step(c, acc):
        xc = hm_ref[0, pl.ds(c * 8, 8)]
        return acc + jnp.sum(xc * xc, axis=0)

    ssq_vec = lax.fori_loop(
        0, NUM_CLASSES // 8, _ssq_step, jnp.zeros((OUT_H, OUT_W), jnp.float32)
    )
    sumsq = jnp.sum(ssq_vec)

    def _zero_step(j, _):
        t_ref[j] = jnp.zeros((OUT_H, OUT_W), jnp.float32)
        return 0

    lax.fori_loop(0, N_BOXES, _zero_step, 0)

    row_iota = lax.broadcasted_iota(jnp.int32, (40, OUT_W), 0)
    col_iota = lax.broadcasted_iota(jnp.int32, (40, OUT_W), 1)
    for i in range(N_BOXES):
        slot = ints_ref[_I_SLOT, b, i]
        rs = ints_ref[_I_RS, b, i]
        cx = ints_ref[_I_CX, b, i]
        cy = ints_ref[_I_CY, b, i]
        rx = ints_ref[_I_RX, b, i]
        ry = ints_ref[_I_RY, b, i]
        denx = flts_ref[_F_DENX, b, i]
        deny = flts_ref[_F_DENY, b, i]
        dy = (rs + row_iota) - cy
        dx = col_iota - cx
        e = dx.astype(jnp.float32) ** 2 / denx + dy.astype(jnp.float32) ** 2 / deny
        g = jnp.exp(-e)
        g = jnp.where(g < _EPS, jnp.float32(0.0), g)
        mask = (jnp.abs(dx) <= rx) & (jnp.abs(dy) <= ry)
        vals = jnp.where(mask, g, jnp.float32(0.0))
        cur = t_ref[slot, pl.ds(rs, 40), :]
        t_ref[slot, pl.ds(rs, 40), :] = jnp.maximum(cur, vals)

    def _corr_step(j, acc):
        lab = ints_ref[_I_LAB, b, j]
        tj = t_ref[j]
        hj = hm_ref[0, lab]
        return acc + tj * (tj - 2.0 * hj)

    corr_vec = lax.fori_loop(
        0, N_BOXES, _corr_step, jnp.zeros((OUT_H, OUT_W), jnp.float32)
    )
    corr = jnp.sum(corr_vec)

    lane = lax.broadcasted_iota(jnp.int32, (1, 2), 1)
    row = jnp.where(lane == 0, sumsq + corr, jnp.float32(0.0))

    @pl.when(b == 0)
    def _():
        out_ref[...] = row

    @pl.when(b > 0)
    def _():
        out_ref[...] = out_ref[...] + row

    @pl.when(b == B - 1)
    def _():
        tot = out_ref[...]
        hm_loss = jnp.sum(jnp.where(lane == 0, tot, 0.0)) / jnp.float32(_NUMEL)
        bl = jnp.float32(0.0)
        for b8 in range(B):
            dsum = jnp.sum(scv_ref[pl.ds(4 * b8, 4), 0])
            csum = jnp.sum(scv_ref[pl.ds(4 * b8, 4), 1])
            bl = bl + dsum / (csum * jnp.float32(4.0))
        box_loss = bl / jnp.float32(B)
        out_ref[...] = jnp.where(lane == 0, hm_loss, box_loss)


def kernel(heatmap, box_2d, boxes, labels):
    ints, flts, idxs4, mavs4 = pl.pallas_call(
        _prep_body,
        out_shape=[
            jax.ShapeDtypeStruct((7, B, N_BOXES), jnp.int32),
            jax.ShapeDtypeStruct((2, B, N_BOXES), jnp.float32),
            jax.ShapeDtypeStruct((B, N_BOXES, 4, 16), jnp.int32),
            jax.ShapeDtypeStruct((B, N_BOXES, 5, 16), jnp.float32),
        ],
    )(boxes, labels)

    idxs = idxs4.reshape(N_WORKERS, 4, 128)
    mavs = mavs4.reshape(N_WORKERS, BOX_PER_W, 5, 16)
    b2flat = box_2d.reshape(B * 4 * OUT_H * OUT_W)

    sc_out = pl.kernel(
        _sc_body,
        out_type=jax.ShapeDtypeStruct((N_WORKERS, 2, 16), jnp.float32),
        mesh=plsc.VectorSubcoreMesh(core_axis_name="c", subcore_axis_name="s"),
        scratch_types=[
            pltpu.VMEM((4, 128), jnp.int32),
            pltpu.VMEM((4, 128), jnp.float32),
            pltpu.VMEM((BOX_PER_W, 5, 16), jnp.float32),
            pltpu.VMEM((2, 16), jnp.float32),
            pltpu.SemaphoreType.DMA,
        ],
    )(b2flat, idxs, mavs)

    tc_out = pl.pallas_call(
        _tc_body,
        grid=(B,),
        in_specs=[
            pl.BlockSpec(memory_space=pltpu.SMEM),
            pl.BlockSpec(memory_space=pltpu.SMEM),
            pl.BlockSpec((1, NUM_CLASSES, OUT_H, OUT_W), lambda b: (b, 0, 0, 0)),
            pl.BlockSpec((N_WORKERS, 2, 16), lambda b: (0, 0, 0)),
        ],
        out_specs=pl.BlockSpec((1, 2), lambda b: (0, 0)),
        out_shape=jax.ShapeDtypeStruct((1, 2), jnp.float32),
        scratch_shapes=[pltpu.VMEM((N_BOXES, OUT_H, OUT_W), jnp.float32)],
        compiler_params=pltpu.CompilerParams(
            dimension_semantics=("arbitrary",),
        ),
    )(ints, flts, heatmap, sc_out)

    return tc_out[0]
